# baseline (speedup 1.0000x reference)
import functools

import jax
import jax.numpy as jnp
from jax.experimental import pallas as pl
from jax.experimental.pallas import tpu as pltpu

B = 16
T = 4096
S = 128
NA = 4
TILE = 512
NT = T // TILE
NEG = -1e30


def _body(len_ref, s_ref, a_ref, w_ref, b_ref, out_ref):
    b = pl.program_id(0)
    k = pl.program_id(1)

    @pl.when(jnp.logical_and(b == 0, k == 0))
    def _init():
        out_ref[0, 0] = 0.0

    x = s_ref[0]
    logits = jnp.dot(x, w_ref[...], preferred_element_type=jnp.float32)
    logits = logits + b_ref[...]
    m = jnp.max(logits, axis=-1, keepdims=True)
    lse = m[:, 0] + jnp.log(jnp.sum(jnp.exp(logits - m), axis=-1))
    acts = a_ref[0]
    cols = jax.lax.broadcasted_iota(jnp.int32, (TILE, S), 1)
    onehot = (cols == acts[0][:, None]).astype(jnp.float32)
    gathered = jnp.sum(logits * onehot, axis=-1)
    t_idx = k * TILE + jax.lax.iota(jnp.int32, TILE)
    mask = (t_idx < len_ref[b]).astype(jnp.float32)
    out_ref[0, 0] += jnp.sum((lse - gathered) * mask)


@jax.jit
def kernel(s_i_batch, actions_batch, lengths, W, bias):
    s4 = s_i_batch[:, :T, :]
    acts3 = actions_batch.reshape(B * NT, 1, TILE).astype(jnp.int32)
    w_pad = jnp.zeros((S, S), jnp.float32).at[:, :NA].set(W)
    b_pad = jnp.full((1, S), NEG, jnp.float32).at[0, :NA].set(bias)

    out = pl.pallas_call(
        _body,
        grid=(B, NT),
        in_specs=[
            pl.BlockSpec(memory_space=pltpu.SMEM),
            pl.BlockSpec((1, TILE, S), lambda b, k: (b, k, 0)),
            pl.BlockSpec((1, 1, TILE), lambda b, k: (b * NT + k, 0, 0)),
            pl.BlockSpec((S, S), lambda b, k: (0, 0)),
            pl.BlockSpec((1, S), lambda b, k: (0, 0)),
        ],
        out_specs=pl.BlockSpec(
            (1, 1), lambda b, k: (0, 0), memory_space=pltpu.SMEM
        ),
        out_shape=jax.ShapeDtypeStruct((1, 1), jnp.float32),
    )(lengths.astype(jnp.int32), s4, acts3, w_pad, b_pad)
    return out[0, 0]


# speedup vs baseline: 1.6095x; 1.6095x over previous
import jax
import jax.numpy as jnp
from jax.experimental import pallas as pl
from jax.experimental.pallas import tpu as pltpu

B = 16
T = 4096
S = 128
NA = 4
AP = 8
TILE = 512
NT = T // TILE
NEG = -1e30


def _body(len_ref, s_ref, a_ref, w_ref, b_ref, out_ref, acc1, acc2):
    b = pl.program_id(0)
    k = pl.program_id(1)

    @pl.when(jnp.logical_and(b == 0, k == 0))
    def _init():
        acc1[...] = jnp.zeros_like(acc1)
        acc2[...] = jnp.zeros_like(acc2)

    @pl.when(k * TILE < len_ref[b])
    def _compute():
        x = s_ref[0]
        lt = jax.lax.dot_general(
            w_ref[...], x, (((0,), (1,)), ((), ())),
            preferred_element_type=jnp.float32,
        ) + b_ref[...]
        m = jnp.max(lt, axis=0, keepdims=True)
        ssum = jnp.sum(jnp.exp(lt - m), axis=0, keepdims=True)
        lse = m + jnp.log(ssum)
        acts = a_ref[0]
        rows = jax.lax.broadcasted_iota(jnp.int32, (AP, TILE), 0)
        t_idx = k * TILE + jax.lax.broadcasted_iota(jnp.int32, (1, TILE), 1)
        mask = (t_idx < len_ref[b]).astype(jnp.float32)
        onehot = jnp.where(rows == acts, mask, 0.0)
        acc1[...] += lse * mask
        acc2[...] += lt * onehot

    @pl.when(jnp.logical_and(b == B - 1, k == NT - 1))
    def _final():
        out_ref[0, 0] = jnp.sum(acc1[...]) - jnp.sum(acc2[...])


@jax.jit
def kernel(s_i_batch, actions_batch, lengths, W, bias):
    s4 = s_i_batch[:, :T, :]
    acts3 = actions_batch.reshape(B * NT, 1, TILE).astype(jnp.int32)
    w_pad = jnp.zeros((S, AP), jnp.float32).at[:, :NA].set(W)
    b_pad = jnp.full((AP, 1), NEG, jnp.float32).at[:NA, 0].set(bias)
    lens = lengths.astype(jnp.int32)

    def clamp(lens, b, k):
        return jnp.minimum(k, pl.cdiv(lens[b], TILE) - 1)

    grid_spec = pltpu.PrefetchScalarGridSpec(
        num_scalar_prefetch=1,
        grid=(B, NT),
        in_specs=[
            pl.BlockSpec((1, TILE, S), lambda b, k, L: (b, clamp(L, b, k), 0)),
            pl.BlockSpec(
                (1, 1, TILE), lambda b, k, L: (b * NT + clamp(L, b, k), 0, 0)
            ),
            pl.BlockSpec((S, AP), lambda b, k, L: (0, 0)),
            pl.BlockSpec((AP, 1), lambda b, k, L: (0, 0)),
        ],
        out_specs=pl.BlockSpec(
            (1, 1), lambda b, k, L: (0, 0), memory_space=pltpu.SMEM
        ),
        scratch_shapes=[
            pltpu.VMEM((1, TILE), jnp.float32),
            pltpu.VMEM((AP, TILE), jnp.float32),
        ],
    )
    out = pl.pallas_call(
        _body,
        grid_spec=grid_spec,
        out_shape=jax.ShapeDtypeStruct((1, 1), jnp.float32),
    )(lens, s4, acts3, w_pad, b_pad)
    return out[0, 0]


# speedup vs baseline: 2.0463x; 1.2714x over previous
import jax
import jax.numpy as jnp
from jax.experimental import pallas as pl
from jax.experimental.pallas import tpu as pltpu

B = 16
T = 4096
S = 128
NA = 4
AP = 8
TILE = 1024
NT = T // TILE
NEG = -1e30


def _body(len_ref, s_ref, a_ref, w_ref, b_ref, out_ref, acc1, acc2):
    b = pl.program_id(0)
    k = pl.program_id(1)

    @pl.when(jnp.logical_and(b == 0, k == 0))
    def _init():
        acc1[...] = jnp.zeros_like(acc1)
        acc2[...] = jnp.zeros_like(acc2)

    @pl.when(k * TILE < len_ref[b])
    def _compute():
        x = s_ref[0]
        lt = jax.lax.dot_general(
            w_ref[...], x, (((0,), (1,)), ((), ())),
            preferred_element_type=jnp.float32,
        ) + b_ref[...]
        m = jnp.max(lt, axis=0, keepdims=True)
        ssum = jnp.sum(jnp.exp(lt - m), axis=0, keepdims=True)
        lse = m + jnp.log(ssum)
        acts = a_ref[0]
        rows = jax.lax.broadcasted_iota(jnp.int32, (AP, TILE), 0)
        t_idx = k * TILE + jax.lax.broadcasted_iota(jnp.int32, (1, TILE), 1)
        mask = (t_idx < len_ref[b]).astype(jnp.float32)
        onehot = jnp.where(rows == acts, mask, 0.0)
        acc1[...] += lse * mask
        acc2[...] += lt * onehot

    @pl.when(jnp.logical_and(b == B - 1, k == NT - 1))
    def _final():
        out_ref[0, 0] = jnp.sum(acc1[...]) - jnp.sum(acc2[...])


@jax.jit
def kernel(s_i_batch, actions_batch, lengths, W, bias):
    s4 = s_i_batch[:, :T, :]
    acts3 = actions_batch.reshape(B * NT, 1, TILE).astype(jnp.int32)
    w_pad = jnp.zeros((S, AP), jnp.float32).at[:, :NA].set(W)
    b_pad = jnp.full((AP, 1), NEG, jnp.float32).at[:NA, 0].set(bias)
    lens = lengths.astype(jnp.int32)

    def clamp(lens, b, k):
        return jnp.minimum(k, pl.cdiv(lens[b], TILE) - 1)

    grid_spec = pltpu.PrefetchScalarGridSpec(
        num_scalar_prefetch=1,
        grid=(B, NT),
        in_specs=[
            pl.BlockSpec((1, TILE, S), lambda b, k, L: (b, clamp(L, b, k), 0)),
            pl.BlockSpec(
                (1, 1, TILE), lambda b, k, L: (b * NT + clamp(L, b, k), 0, 0)
            ),
            pl.BlockSpec((S, AP), lambda b, k, L: (0, 0)),
            pl.BlockSpec((AP, 1), lambda b, k, L: (0, 0)),
        ],
        out_specs=pl.BlockSpec(
            (1, 1), lambda b, k, L: (0, 0), memory_space=pltpu.SMEM
        ),
        scratch_shapes=[
            pltpu.VMEM((1, TILE), jnp.float32),
            pltpu.VMEM((AP, TILE), jnp.float32),
        ],
    )
    out = pl.pallas_call(
        _body,
        grid_spec=grid_spec,
        out_shape=jax.ShapeDtypeStruct((1, 1), jnp.float32),
    )(lens, s4, acts3, w_pad, b_pad)
    return out[0, 0]


# speedup vs baseline: 2.3857x; 1.1658x over previous
import jax
import jax.numpy as jnp
from jax.experimental import pallas as pl
from jax.experimental.pallas import tpu as pltpu

B = 16
T = 4096
S = 128
NA = 4
AP = 8
TILE = 2048
NT = T // TILE
NEG = -1e30


def _body(len_ref, s_ref, a_ref, w_ref, b_ref, out_ref, acc1, acc2):
    b = pl.program_id(0)
    k = pl.program_id(1)

    @pl.when(jnp.logical_and(b == 0, k == 0))
    def _init():
        acc1[...] = jnp.zeros_like(acc1)
        acc2[...] = jnp.zeros_like(acc2)

    @pl.when(k * TILE < len_ref[b])
    def _compute():
        x = s_ref[0]
        lt = jax.lax.dot_general(
            w_ref[...], x, (((0,), (1,)), ((), ())),
            preferred_element_type=jnp.float32,
        ) + b_ref[...]
        m = jnp.max(lt, axis=0, keepdims=True)
        ssum = jnp.sum(jnp.exp(lt - m), axis=0, keepdims=True)
        lse = m + jnp.log(ssum)
        acts = a_ref[0]
        rows = jax.lax.broadcasted_iota(jnp.int32, (AP, TILE), 0)
        t_idx = k * TILE + jax.lax.broadcasted_iota(jnp.int32, (1, TILE), 1)
        mask = (t_idx < len_ref[b]).astype(jnp.float32)
        onehot = jnp.where(rows == acts, mask, 0.0)
        acc1[...] += lse * mask
        acc2[...] += lt * onehot

    @pl.when(jnp.logical_and(b == B - 1, k == NT - 1))
    def _final():
        out_ref[0, 0] = jnp.sum(acc1[...]) - jnp.sum(acc2[...])


@jax.jit
def kernel(s_i_batch, actions_batch, lengths, W, bias):
    s4 = s_i_batch[:, :T, :]
    acts3 = actions_batch.reshape(B * NT, 1, TILE).astype(jnp.int32)
    w_pad = jnp.zeros((S, AP), jnp.float32).at[:, :NA].set(W)
    b_pad = jnp.full((AP, 1), NEG, jnp.float32).at[:NA, 0].set(bias)
    lens = lengths.astype(jnp.int32)

    def clamp(lens, b, k):
        return jnp.minimum(k, pl.cdiv(lens[b], TILE) - 1)

    grid_spec = pltpu.PrefetchScalarGridSpec(
        num_scalar_prefetch=1,
        grid=(B, NT),
        in_specs=[
            pl.BlockSpec((1, TILE, S), lambda b, k, L: (b, clamp(L, b, k), 0)),
            pl.BlockSpec(
                (1, 1, TILE), lambda b, k, L: (b * NT + clamp(L, b, k), 0, 0)
            ),
            pl.BlockSpec((S, AP), lambda b, k, L: (0, 0)),
            pl.BlockSpec((AP, 1), lambda b, k, L: (0, 0)),
        ],
        out_specs=pl.BlockSpec(
            (1, 1), lambda b, k, L: (0, 0), memory_space=pltpu.SMEM
        ),
        scratch_shapes=[
            pltpu.VMEM((1, TILE), jnp.float32),
            pltpu.VMEM((AP, TILE), jnp.float32),
        ],
    )
    out = pl.pallas_call(
        _body,
        grid_spec=grid_spec,
        out_shape=jax.ShapeDtypeStruct((1, 1), jnp.float32),
    )(lens, s4, acts3, w_pad, b_pad)
    return out[0, 0]


# speedup vs baseline: 2.7784x; 1.1646x over previous
import jax
import jax.numpy as jnp
from jax.experimental import pallas as pl
from jax.experimental.pallas import tpu as pltpu

B = 16
T = 4096
S = 128
NA = 4
AP = 8
TILE = 4096
NT = T // TILE
NEG = -1e30


def _body(len_ref, s_ref, a_ref, w_ref, b_ref, out_ref, acc1, acc2):
    b = pl.program_id(0)
    k = pl.program_id(1)

    @pl.when(jnp.logical_and(b == 0, k == 0))
    def _init():
        acc1[...] = jnp.zeros_like(acc1)
        acc2[...] = jnp.zeros_like(acc2)

    @pl.when(k * TILE < len_ref[b])
    def _compute():
        x = s_ref[0]
        lt = jax.lax.dot_general(
            w_ref[...], x, (((0,), (1,)), ((), ())),
            preferred_element_type=jnp.float32,
        ) + b_ref[...]
        m = jnp.max(lt, axis=0, keepdims=True)
        ssum = jnp.sum(jnp.exp(lt - m), axis=0, keepdims=True)
        lse = m + jnp.log(ssum)
        acts = a_ref[0]
        rows = jax.lax.broadcasted_iota(jnp.int32, (AP, TILE), 0)
        t_idx = k * TILE + jax.lax.broadcasted_iota(jnp.int32, (1, TILE), 1)
        mask = (t_idx < len_ref[b]).astype(jnp.float32)
        onehot = jnp.where(rows == acts, mask, 0.0)
        acc1[...] += lse * mask
        acc2[...] += lt * onehot

    @pl.when(jnp.logical_and(b == B - 1, k == NT - 1))
    def _final():
        out_ref[0, 0] = jnp.sum(acc1[...]) - jnp.sum(acc2[...])


@jax.jit
def kernel(s_i_batch, actions_batch, lengths, W, bias):
    s4 = s_i_batch[:, :T, :]
    acts3 = actions_batch.reshape(B * NT, 1, TILE).astype(jnp.int32)
    w_pad = jnp.zeros((S, AP), jnp.float32).at[:, :NA].set(W)
    b_pad = jnp.full((AP, 1), NEG, jnp.float32).at[:NA, 0].set(bias)
    lens = lengths.astype(jnp.int32)

    def clamp(lens, b, k):
        return jnp.minimum(k, pl.cdiv(lens[b], TILE) - 1)

    grid_spec = pltpu.PrefetchScalarGridSpec(
        num_scalar_prefetch=1,
        grid=(B, NT),
        in_specs=[
            pl.BlockSpec((1, TILE, S), lambda b, k, L: (b, clamp(L, b, k), 0)),
            pl.BlockSpec(
                (1, 1, TILE), lambda b, k, L: (b * NT + clamp(L, b, k), 0, 0)
            ),
            pl.BlockSpec((S, AP), lambda b, k, L: (0, 0)),
            pl.BlockSpec((AP, 1), lambda b, k, L: (0, 0)),
        ],
        out_specs=pl.BlockSpec(
            (1, 1), lambda b, k, L: (0, 0), memory_space=pltpu.SMEM
        ),
        scratch_shapes=[
            pltpu.VMEM((1, TILE), jnp.float32),
            pltpu.VMEM((AP, TILE), jnp.float32),
        ],
    )
    out = pl.pallas_call(
        _body,
        grid_spec=grid_spec,
        out_shape=jax.ShapeDtypeStruct((1, 1), jnp.float32),
    )(lens, s4, acts3, w_pad, b_pad)
    return out[0, 0]
